# baseline (speedup 1.0000x reference)
import functools

import jax
import jax.numpy as jnp
from jax import lax
from jax.experimental import pallas as pl
from jax.experimental.pallas import tpu as pltpu
from jax.experimental.pallas import tpu_sc as plsc

COUNT = 8
L1 = 1024
L2 = 15
L3 = 32
G = L2 + 1
SQR_C = 0.9921875

TB = 2048

def _tc_body(x_ref, w1_ref, w1f_ref, b1_ref, b1f_ref,
             w2big_ref, b2_ref, w3big_ref, b3_ref,
             out_ref, weff_ref, beff_ref, w2b_ref, w3b_ref):
    @pl.when(pl.program_id(0) == 0)
    def _():
        w1f_big = jnp.concatenate([w1f_ref[...]] * COUNT, axis=0)
        weff = w1_ref[...] + w1f_big
        weff_ref[...] = weff.astype(jnp.bfloat16)
        b1f_big = jnp.concatenate([b1f_ref[...]] * COUNT, axis=1)
        ones = jnp.ones((8, L1), jnp.float32)
        corr = jax.lax.dot_general(ones, weff, (((1,), (1,)), ((), ())),
                                   preferred_element_type=jnp.float32)
        beff_ref[...] = (b1_ref[...] + b1f_big + 0.5 * corr[:1, :])
        w2b_ref[...] = w2big_ref[...].astype(jnp.bfloat16)
        w3b_ref[...] = w3big_ref[...].astype(jnp.bfloat16)

    xb = x_ref[...]
    xc = (xb - 0.5).astype(jnp.bfloat16)
    y = jax.lax.dot_general(xc, weff_ref[...], (((1,), (1,)), ((), ())),
                            preferred_element_type=jnp.float32) + beff_ref[...]
    act = jnp.concatenate(
        [jnp.clip(y * y * SQR_C, 0.0, 1.0), jnp.clip(y, 0.0, 1.0)],
        axis=1).astype(jnp.bfloat16)
    l2 = jax.lax.dot_general(act, w2b_ref[...], (((1,), (0,)), ((), ())),
                             preferred_element_type=jnp.float32) + b2_ref[...]
    l2x = jnp.clip(l2, 0.0, 1.0).astype(jnp.bfloat16)
    l3 = jax.lax.dot_general(l2x, w3b_ref[...], (((1,), (0,)), ((), ())),
                             preferred_element_type=jnp.float32)
    skip = jnp.concatenate(
        [y[:, G * k + L2:G * k + G] for k in range(COUNT)], axis=1)
    out_ref[...] = (l3 + skip + b3_ref[...]).T


def _tc_scores(x, W1r, W1f, b1r, b1fr, W2big, b2r, W3big, b3r):
    B = x.shape[0]
    full = lambda i: (0, 0)
    return pl.pallas_call(
        _tc_body,
        grid=(B // TB,),
        in_specs=[
            pl.BlockSpec((TB, L1), lambda i: (i, 0)),
            pl.BlockSpec((COUNT * G, L1), full),
            pl.BlockSpec((G, L1), full),
            pl.BlockSpec((1, COUNT * G), full),
            pl.BlockSpec((1, G), full),
            pl.BlockSpec((2 * COUNT * G, COUNT * L3), full),
            pl.BlockSpec((1, COUNT * L3), full),
            pl.BlockSpec((COUNT * L3, COUNT), full),
            pl.BlockSpec((1, COUNT), full),
        ],
        out_specs=pl.BlockSpec((COUNT, TB), lambda i: (0, i)),
        out_shape=jax.ShapeDtypeStruct((COUNT, B), jnp.float32),
        scratch_shapes=[
            pltpu.VMEM((COUNT * G, L1), jnp.bfloat16),
            pltpu.VMEM((1, COUNT * G), jnp.float32),
            pltpu.VMEM((2 * COUNT * G, COUNT * L3), jnp.bfloat16),
            pltpu.VMEM((COUNT * L3, COUNT), jnp.bfloat16),
        ],
        compiler_params=pltpu.CompilerParams(
            dimension_semantics=("arbitrary",),
        ),
    )(x, W1r, W1f, b1r, b1fr, W2big, b2r, W3big, b3r)


def _make_sc_gather(B):
    NC, NS, L = 2, 16, 16
    NW = NC * NS
    bpw = B // NW
    mesh = plsc.VectorSubcoreMesh(core_axis_name="c", subcore_axis_name="s")

    CH = 128
    NCH = bpw // CH

    @functools.partial(
        pl.kernel, mesh=mesh,
        out_type=jax.ShapeDtypeStruct((B,), jnp.float32),
        scratch_types=[
            pltpu.VMEM((bpw,), jnp.int32),
            pltpu.VMEM((NCH, CH), jnp.int32),
            pltpu.VMEM((bpw,), jnp.float32),
            pltpu.SemaphoreType.DMA,
        ],
    )
    def sc_gather(scores_hbm, idx_hbm, out_hbm, idx_v, fidx_v, out_v, sem):
        wid = lax.axis_index("s") * NC + lax.axis_index("c")
        base = wid * bpw
        pltpu.sync_copy(idx_hbm.at[pl.ds(base, bpw)], idx_v)
        per_row = CH // L
        for i in range(bpw // L):
            lane = lax.iota(jnp.int32, L)
            flat = idx_v[pl.ds(i * L, L)] * B + (lane + (base + i * L))
            fidx_v[i // per_row, pl.ds((i % per_row) * L, L)] = flat
        for j in range(NCH):
            pltpu.async_copy(scores_hbm.at[fidx_v.at[j]],
                             out_v.at[pl.ds(j * CH, CH)], sem).wait()
        pltpu.sync_copy(out_v, out_hbm.at[pl.ds(base, bpw)])

    return sc_gather


def kernel(x, ls_indices, W1, b1, W1f, b1f, W2, b2, W3, b3):
    B = x.shape[0]
    idx = ls_indices.astype(jnp.int32)
    W1r = W1.reshape(COUNT * G, L1)
    b1r = b1.reshape(1, COUNT * G)
    b1fr = b1f.reshape(1, G)
    eye = jnp.eye(COUNT, dtype=W2.dtype)
    W2sqr = jnp.pad(jnp.transpose(W2[:, :, :L2], (0, 2, 1)),
                    ((0, 0), (0, 1), (0, 0)))
    W2raw = jnp.pad(jnp.transpose(W2[:, :, L2:], (0, 2, 1)),
                    ((0, 0), (0, 1), (0, 0)))
    top = jnp.einsum('kjo,kK->kjKo', W2sqr, eye).reshape(COUNT * G, COUNT * L3)
    bot = jnp.einsum('kjo,kK->kjKo', W2raw, eye).reshape(COUNT * G, COUNT * L3)
    W2big = jnp.concatenate([top, bot], axis=0)
    b2r = b2.reshape(1, COUNT * L3)
    W3big = jnp.einsum('ko,kK->koK', W3.reshape(COUNT, L3),
                       eye).reshape(COUNT * L3, COUNT)
    b3r = b3.reshape(1, COUNT)

    scores = _tc_scores(x, W1r, W1f, b1r, b1fr, W2big, b2r, W3big, b3r)
    out = _make_sc_gather(B)(scores.reshape(COUNT * B), idx)
    return out.reshape(B, 1)


# speedup vs baseline: 1.0691x; 1.0691x over previous
import functools

import jax
import jax.numpy as jnp
from jax import lax
from jax.experimental import pallas as pl
from jax.experimental.pallas import tpu as pltpu
from jax.experimental.pallas import tpu_sc as plsc

COUNT = 8
L1 = 1024
L2 = 15
L3 = 32
G = L2 + 1
SQR_C = 0.9921875

TB = 2048

def _tc_body(x_ref, x2_ref, w1_ref, w1f_ref, b1_ref, b1f_ref,
             w2big_ref, b2_ref, w3big_ref, b3_ref,
             out_ref, weff_ref, beff_ref, w2b_ref, w3b_ref):
    @pl.when(pl.program_id(0) == 0)
    def _():
        w1f_big = jnp.concatenate([w1f_ref[...]] * COUNT, axis=0)
        weff = w1_ref[...] + w1f_big
        weff_ref[...] = weff.astype(jnp.bfloat16)
        b1f_big = jnp.concatenate([b1f_ref[...]] * COUNT, axis=1)
        ones = jnp.ones((8, L1), jnp.float32)
        corr = jax.lax.dot_general(ones, weff, (((1,), (1,)), ((), ())),
                                   preferred_element_type=jnp.float32)
        beff_ref[...] = (b1_ref[...] + b1f_big + 0.5 * corr[:1, :])
        w2b_ref[...] = w2big_ref[...].astype(jnp.bfloat16)
        w3b_ref[...] = w3big_ref[...].astype(jnp.bfloat16)

    H = L1 // 2
    xcl = (x_ref[...] - 0.5).astype(jnp.bfloat16)
    xcr = (x2_ref[...] - 0.5).astype(jnp.bfloat16)
    yl = jax.lax.dot_general(xcl, weff_ref[:, :H], (((1,), (1,)), ((), ())),
                             preferred_element_type=jnp.float32)
    yr = jax.lax.dot_general(xcr, weff_ref[:, H:], (((1,), (1,)), ((), ())),
                             preferred_element_type=jnp.float32)
    y = yl + yr + beff_ref[...]
    act = jnp.concatenate(
        [jnp.clip(y * y * SQR_C, 0.0, 1.0), jnp.clip(y, 0.0, 1.0)],
        axis=1).astype(jnp.bfloat16)
    l2 = jax.lax.dot_general(act, w2b_ref[...], (((1,), (0,)), ((), ())),
                             preferred_element_type=jnp.float32) + b2_ref[...]
    l2x = jnp.clip(l2, 0.0, 1.0).astype(jnp.bfloat16)
    l3 = jax.lax.dot_general(l2x, w3b_ref[...], (((1,), (0,)), ((), ())),
                             preferred_element_type=jnp.float32)
    skip = jnp.concatenate(
        [y[:, G * k + L2:G * k + G] for k in range(COUNT)], axis=1)
    out_ref[...] = (l3 + skip + b3_ref[...]).T


def _tc_scores(x, W1r, W1f, b1r, b1fr, W2big, b2r, W3big, b3r):
    B = x.shape[0]
    full = lambda i: (0, 0)
    return pl.pallas_call(
        _tc_body,
        grid=(B // TB,),
        in_specs=[
            pl.BlockSpec((TB, L1 // 2), lambda i: (i, 0)),
            pl.BlockSpec((TB, L1 // 2), lambda i: (i, 1)),
            pl.BlockSpec((COUNT * G, L1), full),
            pl.BlockSpec((G, L1), full),
            pl.BlockSpec((1, COUNT * G), full),
            pl.BlockSpec((1, G), full),
            pl.BlockSpec((2 * COUNT * G, COUNT * L3), full),
            pl.BlockSpec((1, COUNT * L3), full),
            pl.BlockSpec((COUNT * L3, COUNT), full),
            pl.BlockSpec((1, COUNT), full),
        ],
        out_specs=pl.BlockSpec((COUNT, TB), lambda i: (0, i)),
        out_shape=jax.ShapeDtypeStruct((COUNT, B), jnp.float32),
        scratch_shapes=[
            pltpu.VMEM((COUNT * G, L1), jnp.bfloat16),
            pltpu.VMEM((1, COUNT * G), jnp.float32),
            pltpu.VMEM((2 * COUNT * G, COUNT * L3), jnp.bfloat16),
            pltpu.VMEM((COUNT * L3, COUNT), jnp.bfloat16),
        ],
        compiler_params=pltpu.CompilerParams(
            dimension_semantics=("arbitrary",),
        ),
    )(x, x, W1r, W1f, b1r, b1fr, W2big, b2r, W3big, b3r)


def _make_sc_gather(B):
    NC, NS, L = 2, 16, 16
    NW = NC * NS
    bpw = B // NW
    mesh = plsc.VectorSubcoreMesh(core_axis_name="c", subcore_axis_name="s")

    CH = 128
    NCH = bpw // CH

    @functools.partial(
        pl.kernel, mesh=mesh,
        out_type=jax.ShapeDtypeStruct((B,), jnp.float32),
        scratch_types=[
            pltpu.VMEM((bpw,), jnp.int32),
            pltpu.VMEM((NCH, CH), jnp.int32),
            pltpu.VMEM((bpw,), jnp.float32),
            pltpu.SemaphoreType.DMA,
        ],
    )
    def sc_gather(scores_hbm, idx_hbm, out_hbm, idx_v, fidx_v, out_v, sem):
        wid = lax.axis_index("s") * NC + lax.axis_index("c")
        base = wid * bpw
        pltpu.sync_copy(idx_hbm.at[pl.ds(base, bpw)], idx_v)
        per_row = CH // L
        for i in range(bpw // L):
            lane = lax.iota(jnp.int32, L)
            flat = idx_v[pl.ds(i * L, L)] * B + (lane + (base + i * L))
            fidx_v[i // per_row, pl.ds((i % per_row) * L, L)] = flat
        for j in range(NCH):
            pltpu.async_copy(scores_hbm.at[fidx_v.at[j]],
                             out_v.at[pl.ds(j * CH, CH)], sem).wait()
        pltpu.sync_copy(out_v, out_hbm.at[pl.ds(base, bpw)])

    return sc_gather


def kernel(x, ls_indices, W1, b1, W1f, b1f, W2, b2, W3, b3):
    B = x.shape[0]
    idx = ls_indices.astype(jnp.int32)
    W1r = W1.reshape(COUNT * G, L1)
    b1r = b1.reshape(1, COUNT * G)
    b1fr = b1f.reshape(1, G)
    eye = jnp.eye(COUNT, dtype=W2.dtype)
    W2sqr = jnp.pad(jnp.transpose(W2[:, :, :L2], (0, 2, 1)),
                    ((0, 0), (0, 1), (0, 0)))
    W2raw = jnp.pad(jnp.transpose(W2[:, :, L2:], (0, 2, 1)),
                    ((0, 0), (0, 1), (0, 0)))
    top = jnp.einsum('kjo,kK->kjKo', W2sqr, eye).reshape(COUNT * G, COUNT * L3)
    bot = jnp.einsum('kjo,kK->kjKo', W2raw, eye).reshape(COUNT * G, COUNT * L3)
    W2big = jnp.concatenate([top, bot], axis=0)
    b2r = b2.reshape(1, COUNT * L3)
    W3big = jnp.einsum('ko,kK->koK', W3.reshape(COUNT, L3),
                       eye).reshape(COUNT * L3, COUNT)
    b3r = b3.reshape(1, COUNT)

    scores = _tc_scores(x, W1r, W1f, b1r, b1fr, W2big, b2r, W3big, b3r)
    out = _make_sc_gather(B)(scores.reshape(COUNT * B), idx)
    return out.reshape(B, 1)


# speedup vs baseline: 1.1108x; 1.0390x over previous
import functools

import jax
import jax.numpy as jnp
from jax import lax
from jax.experimental import pallas as pl
from jax.experimental.pallas import tpu as pltpu
from jax.experimental.pallas import tpu_sc as plsc

COUNT = 8
L1 = 1024
L2 = 15
L3 = 32
G = L2 + 1
SQR_C = 0.9921875

TB = 2048

def _tc_body(x_ref, x2_ref, w1_ref, w1f_ref, b1_ref, b1f_ref,
             w2big_ref, b2_ref, w3big_ref, b3_ref,
             out_ref, weff_ref, beff_ref, w2b_ref, w3b_ref):
    @pl.when(pl.program_id(0) == 0)
    def _():
        w1f_big = jnp.concatenate([w1f_ref[...]] * COUNT, axis=0)
        weff = w1_ref[...] + w1f_big
        weff_ref[...] = weff.astype(jnp.bfloat16)
        b1f_big = jnp.concatenate([b1f_ref[...]] * COUNT, axis=1)
        ones = jnp.ones((8, L1), jnp.float32)
        corr = jax.lax.dot_general(ones, weff, (((1,), (1,)), ((), ())),
                                   preferred_element_type=jnp.float32)
        beff_ref[...] = (b1_ref[...] + b1f_big + 0.5 * corr[:1, :])
        w2t = w2big_ref[...].T
        zrow = jnp.zeros((1, COUNT * L3), jnp.float32)
        blk_sqr = jnp.concatenate([w2t[:L2], zrow], axis=0)
        blk_raw = jnp.concatenate([w2t[L2:], zrow], axis=0)
        rid = jax.lax.broadcasted_iota(jnp.int32, (COUNT * G, COUNT * L3), 0)
        cid = jax.lax.broadcasted_iota(jnp.int32, (COUNT * G, COUNT * L3), 1)
        mask = (rid // G) == (cid // L3)
        tiled_sqr = jnp.concatenate([blk_sqr] * COUNT, axis=0)
        tiled_raw = jnp.concatenate([blk_raw] * COUNT, axis=0)
        w2b_ref[...] = jnp.concatenate(
            [jnp.where(mask, tiled_sqr, 0.0),
             jnp.where(mask, tiled_raw, 0.0)], axis=0).astype(jnp.bfloat16)
        w3t = w3big_ref[...].T
        rid3 = jax.lax.broadcasted_iota(jnp.int32, (COUNT * L3, COUNT), 0)
        cid3 = jax.lax.broadcasted_iota(jnp.int32, (COUNT * L3, COUNT), 1)
        w3b_ref[...] = jnp.where(
            rid3 // L3 == cid3,
            jnp.concatenate([w3t] * COUNT, axis=0), 0.0).astype(jnp.bfloat16)

    H = L1 // 2
    xcl = (x_ref[...] - 0.5).astype(jnp.bfloat16)
    xcr = (x2_ref[...] - 0.5).astype(jnp.bfloat16)
    yl = jax.lax.dot_general(xcl, weff_ref[:, :H], (((1,), (1,)), ((), ())),
                             preferred_element_type=jnp.float32)
    yr = jax.lax.dot_general(xcr, weff_ref[:, H:], (((1,), (1,)), ((), ())),
                             preferred_element_type=jnp.float32)
    y = yl + yr + beff_ref[...]
    act = jnp.concatenate(
        [jnp.clip(y * y * SQR_C, 0.0, 1.0), jnp.clip(y, 0.0, 1.0)],
        axis=1).astype(jnp.bfloat16)
    l2 = jax.lax.dot_general(act, w2b_ref[...], (((1,), (0,)), ((), ())),
                             preferred_element_type=jnp.float32) + b2_ref[...]
    l2x = jnp.clip(l2, 0.0, 1.0).astype(jnp.bfloat16)
    l3 = jax.lax.dot_general(l2x, w3b_ref[...], (((1,), (0,)), ((), ())),
                             preferred_element_type=jnp.float32)
    skip = jnp.concatenate(
        [y[:, G * k + L2:G * k + G] for k in range(COUNT)], axis=1)
    out_ref[...] = (l3 + skip + b3_ref[...]).T


def _tc_scores(x, W1r, W1f, b1r, b1fr, W2big, b2r, W3big, b3r):
    B = x.shape[0]
    full = lambda i: (0, 0)
    return pl.pallas_call(
        _tc_body,
        grid=(B // TB,),
        in_specs=[
            pl.BlockSpec((TB, L1 // 2), lambda i: (i, 0)),
            pl.BlockSpec((TB, L1 // 2), lambda i: (i, 1)),
            pl.BlockSpec((COUNT * G, L1), full),
            pl.BlockSpec((G, L1), full),
            pl.BlockSpec((1, COUNT * G), full),
            pl.BlockSpec((1, G), full),
            pl.BlockSpec((COUNT * L3, 2 * L2), full),
            pl.BlockSpec((1, COUNT * L3), full),
            pl.BlockSpec((COUNT, L3), full),
            pl.BlockSpec((1, COUNT), full),
        ],
        out_specs=pl.BlockSpec((COUNT, TB), lambda i: (0, i)),
        out_shape=jax.ShapeDtypeStruct((COUNT, B), jnp.float32),
        scratch_shapes=[
            pltpu.VMEM((COUNT * G, L1), jnp.bfloat16),
            pltpu.VMEM((1, COUNT * G), jnp.float32),
            pltpu.VMEM((2 * COUNT * G, COUNT * L3), jnp.bfloat16),
            pltpu.VMEM((COUNT * L3, COUNT), jnp.bfloat16),
        ],
        compiler_params=pltpu.CompilerParams(
            dimension_semantics=("arbitrary",),
        ),
    )(x, x, W1r, W1f, b1r, b1fr, W2big, b2r, W3big, b3r)


def _make_sc_gather(B):
    NC, NS, L = 2, 16, 16
    NW = NC * NS
    bpw = B // NW
    mesh = plsc.VectorSubcoreMesh(core_axis_name="c", subcore_axis_name="s")

    CH = 128
    NCH = bpw // CH

    @functools.partial(
        pl.kernel, mesh=mesh,
        out_type=jax.ShapeDtypeStruct((B,), jnp.float32),
        scratch_types=[
            pltpu.VMEM((bpw,), jnp.int32),
            pltpu.VMEM((NCH, CH), jnp.int32),
            pltpu.VMEM((bpw,), jnp.float32),
            pltpu.SemaphoreType.DMA,
        ],
    )
    def sc_gather(scores_hbm, idx_hbm, out_hbm, idx_v, fidx_v, out_v, sem):
        wid = lax.axis_index("s") * NC + lax.axis_index("c")
        base = wid * bpw
        pltpu.sync_copy(idx_hbm.at[pl.ds(base, bpw)], idx_v)
        per_row = CH // L
        for i in range(bpw // L):
            lane = lax.iota(jnp.int32, L)
            flat = idx_v[pl.ds(i * L, L)] * B + (lane + (base + i * L))
            fidx_v[i // per_row, pl.ds((i % per_row) * L, L)] = flat
        for j in range(NCH):
            pltpu.async_copy(scores_hbm.at[fidx_v.at[j]],
                             out_v.at[pl.ds(j * CH, CH)], sem).wait()
        pltpu.sync_copy(out_v, out_hbm.at[pl.ds(base, bpw)])

    return sc_gather


def kernel(x, ls_indices, W1, b1, W1f, b1f, W2, b2, W3, b3):
    B = x.shape[0]
    idx = ls_indices.astype(jnp.int32)
    W1r = W1.reshape(COUNT * G, L1)
    b1r = b1.reshape(1, COUNT * G)
    b1fr = b1f.reshape(1, G)
    W2r = W2.reshape(COUNT * L3, 2 * L2)
    b2r = b2.reshape(1, COUNT * L3)
    W3r = W3.reshape(COUNT, L3)
    b3r = b3.reshape(1, COUNT)

    scores = _tc_scores(x, W1r, W1f, b1r, b1fr, W2r, b2r, W3r, b3r)
    out = _make_sc_gather(B)(scores.reshape(COUNT * B), idx)
    return out.reshape(B, 1)
